# baseline (speedup 1.0000x reference)
import functools

import jax
import jax.numpy as jnp
from jax import lax
from jax.experimental import pallas as pl
from jax.experimental.pallas import tpu as pltpu
from jax.experimental.pallas import tpu_sc as plsc

N = 4_000_000
NN = 2_000_000
NM = 1_800_000
MAGIC = 12582912.0
QMAX = 2047.0

NC, NS, L = 2, 16, 16
NW = NC * NS

UC = 14_336
UW = UC // 4
NG = UC // 64
MOV_LO = NN // UC
MOV_HI = (NN + NM - 1) // UC
MASK_FRAME = (MOV_HI - MOV_LO + 1) * UC
MASK_LEFT_PAD = NN - MOV_LO * UC
NFULL = N // UC
TAIL = N - NFULL * UC
NV = -(-NFULL // NW)
NBUF = 3


def _sc_quantize_body(pos_hbm, mwords_hbm, out_hbm,
                      shared, tbuf, tob, mbufs, isems, osems, tsem):
    cid = lax.axis_index("c")
    sid = lax.axis_index("s")
    wid = sid * NC + cid

    def chunk_id(v):
        return wid + v * NW

    def is_mov(k):
        return jnp.logical_and(k >= MOV_LO, k <= MOV_HI)

    def sp_slot(v):
        b = v % NBUF
        return shared.at[pl.ds((sid * NBUF + b) * UC, UC)]

    def descs(v):
        k = chunk_id(v)
        b = v % NBUF
        row = sp_slot(v)
        in_d = pltpu.make_async_copy(
            pos_hbm.at[pl.ds(k * UC, UC)], row, isems[b])
        m_d = pltpu.make_async_copy(
            mwords_hbm.at[pl.ds((k - MOV_LO) * UW, UW)], mbufs[b], isems[b])
        out_d = pltpu.make_async_copy(
            row, out_hbm.at[pl.ds(k * UC, UC)], osems[b])
        return k, in_d, m_d, out_d

    def issue_in(v):
        k, in_d, m_d, _ = descs(v)

        @pl.when(k < NFULL)
        def _():
            in_d.start()

            @pl.when(is_mov(k))
            def _():
                m_d.start()

    def quantize(v):
        mbuf = mbufs[v % NBUF]

        def grp(g, c):
            w16 = mbuf[pl.ds(g * 16, 16)]
            for j in range(4):
                sl = pl.ds(g * 64 + j * 16, 16)
                x = tbuf[sl]
                mb = (w16 << (31 - 8 * j)) < 0
                q = jnp.minimum((x + MAGIC) - MAGIC, QMAX)
                tob[sl] = jnp.where(mb, q, x)
            return c

        lax.fori_loop(0, NG, grp, 0)

    for v in range(NBUF):
        issue_in(v)
    for v in range(NV):
        k, in_d, m_d, out_d = descs(v)
        if v >= NBUF:
            kp, _, _, out_dp = descs(v - NBUF)

            @pl.when(kp < NFULL)
            def _():
                out_dp.wait()

        @pl.when(k < NFULL)
        def _():
            in_d.wait()

            @pl.when(is_mov(k))
            def _(v=v):
                m_d.wait()
                row = sp_slot(v)
                pltpu.async_copy(row, tbuf, tsem).wait()
                quantize(v)
                pltpu.async_copy(tob, row, tsem).wait()

            out_d.start()
        if v + NBUF < NV:
            issue_in(v + NBUF)
    for v in range(NV - NBUF, NV):
        k, _, _, out_d = descs(v)

        @pl.when(k < NFULL)
        def _():
            out_d.wait()

    @pl.when(wid == 0)
    def _tail():
        tb = tbuf.at[pl.ds(0, TAIL)]
        pltpu.sync_copy(pos_hbm.at[pl.ds(NFULL * UC, TAIL)], tb)
        pltpu.sync_copy(tb, out_hbm.at[pl.ds(NFULL * UC, TAIL)])


@functools.lru_cache(maxsize=1)
def _build():
    mesh = plsc.VectorSubcoreMesh(core_axis_name="c", subcore_axis_name="s",
                                  num_cores=NC, num_subcores=NS)
    return pl.kernel(
        _sc_quantize_body,
        out_type=jax.ShapeDtypeStruct((N,), jnp.float32),
        mesh=mesh,
        scratch_types=[
            pltpu.VMEM_SHARED((NS * NBUF * UC,), jnp.float32),
            pltpu.VMEM((UC,), jnp.float32),
            pltpu.VMEM((UC,), jnp.float32),
            [pltpu.VMEM((UW,), jnp.int32) for _ in range(NBUF)],
            [pltpu.SemaphoreType.DMA for _ in range(NBUF)],
            [pltpu.SemaphoreType.DMA for _ in range(NBUF)],
            pltpu.SemaphoreType.DMA,
        ],
        compiler_params=pltpu.CompilerParams(needs_layout_passes=False),
    )


def kernel(pos, mask):
    maskp = jnp.concatenate([
        jnp.zeros((MASK_LEFT_PAD,), jnp.uint8),
        mask.view(jnp.uint8),
        jnp.zeros((MASK_FRAME - MASK_LEFT_PAD - NM,), jnp.uint8),
    ])
    mwords = (maskp.reshape(-1, 4, 16).transpose(0, 2, 1)
              .reshape(-1).view(jnp.int32))
    return _build()(pos, mwords)


# speedup vs baseline: 2.9440x; 2.9440x over previous
import functools

import jax
import jax.numpy as jnp
from jax import lax
from jax.experimental import pallas as pl
from jax.experimental.pallas import tpu as pltpu
from jax.experimental.pallas import tpu_sc as plsc

N = 4_000_000
NN = 2_000_000
NM = 1_800_000
NROWS = 2048

LANES = 128
R = 744
POS_ROWS = N // LANES
FRAME_LO = 1_999_872
FRAME_LEN = 1_809_408
FRAME_HI = FRAME_LO + FRAME_LEN
FRAME_ROWS = FRAME_LEN // LANES
FRAME_BLKS = 19
FRAME_BLK0 = 21
MASK_LEFT_PAD = NN - FRAME_LO

NC, NS = 2, 16
NW = NC * NS

CC = 15_872
NCH = N // CC
B_LO = FRAME_LO // CC
B_HI = FRAME_HI // CC
TAIL = N - NCH * CC
NV = -(-NCH // NW)
NBUF = 4


def _sc_assemble_body(pos_hbm, frame_hbm, out_hbm, shared, isems, osems):
    cid = lax.axis_index("c")
    sid = lax.axis_index("s")
    wid = sid * NC + cid

    def descs(u):
        j = wid + u * NW
        b = u % NBUF
        in_frame = jnp.logical_and(j >= B_LO, j < B_HI)
        row = shared.at[pl.ds((sid * NBUF + b) * CC, CC)]
        inp_d = pltpu.make_async_copy(
            pos_hbm.at[pl.ds(j * CC, CC)], row, isems[b])
        inf_d = pltpu.make_async_copy(
            frame_hbm.at[pl.ds(j * CC - FRAME_LO, CC)], row, isems[b])
        out_d = pltpu.make_async_copy(
            row, out_hbm.at[pl.ds(j * CC, CC)], osems[b])
        return j, in_frame, inp_d, inf_d, out_d

    def issue_in(u):
        j, in_frame, inp_d, inf_d, _ = descs(u)

        @pl.when(jnp.logical_and(j < NCH, in_frame))
        def _():
            inf_d.start()

        @pl.when(jnp.logical_and(j < NCH, jnp.logical_not(in_frame)))
        def _():
            inp_d.start()

    issue_in(0)
    for u in range(NV):
        j, _, inp_d, _, out_d = descs(u)
        if u >= NBUF - 1:
            jp, _, _, _, out_dp = descs(u - (NBUF - 1))

            @pl.when(jp < NCH)
            def _():
                out_dp.wait()
        if u + 1 < NV:
            issue_in(u + 1)

        @pl.when(j < NCH)
        def _():
            inp_d.wait()
            out_d.start()
    for u in range(max(NV - (NBUF - 1), 0), NV):
        j, _, _, _, out_d = descs(u)

        @pl.when(j < NCH)
        def _():
            out_d.wait()

    @pl.when(wid == 0)
    def _tail():
        tb = shared.at[pl.ds(0, TAIL)]
        pltpu.sync_copy(pos_hbm.at[pl.ds(NCH * CC, TAIL)], tb)
        pltpu.sync_copy(tb, out_hbm.at[pl.ds(NCH * CC, TAIL)])


@functools.lru_cache(maxsize=1)
def _build_sc():
    mesh = plsc.VectorSubcoreMesh(core_axis_name="c", subcore_axis_name="s",
                                  num_cores=NC, num_subcores=NS)
    return pl.kernel(
        _sc_assemble_body,
        out_type=jax.ShapeDtypeStruct((N,), jnp.float32),
        mesh=mesh,
        scratch_types=[
            pltpu.VMEM_SHARED((NS * NBUF * CC,), jnp.float32),
            [pltpu.SemaphoreType.DMA for _ in range(NBUF)],
            [pltpu.SemaphoreType.DMA for _ in range(NBUF)],
        ],
        compiler_params=pltpu.CompilerParams(needs_layout_passes=False),
    )


def _tc_quant_body(pos_ref, mask_ref, out_ref):
    x = pos_ref[...]
    q = jnp.clip(jnp.round(x), 0.0, float(NROWS - 1))
    out_ref[...] = jnp.where(mask_ref[...], q, x)


def kernel(pos, mask):
    pos2 = pos.reshape(POS_ROWS, LANES)
    maskp = jnp.concatenate([
        jnp.zeros((MASK_LEFT_PAD,), jnp.bool_),
        mask,
        jnp.zeros((FRAME_LEN - MASK_LEFT_PAD - NM,), jnp.bool_),
    ]).reshape(FRAME_ROWS, LANES)

    new_yf = pl.pallas_call(
        _tc_quant_body,
        grid=(FRAME_BLKS,),
        in_specs=[
            pl.BlockSpec((R, LANES), lambda i: (i + FRAME_BLK0, 0)),
            pl.BlockSpec((R, LANES), lambda i: (i, 0)),
        ],
        out_specs=pl.BlockSpec((R, LANES), lambda i: (i, 0)),
        out_shape=jax.ShapeDtypeStruct((FRAME_ROWS, LANES), jnp.float32),
    )(pos2, maskp)

    return _build_sc()(pos, new_yf.reshape(FRAME_LEN))


# speedup vs baseline: 5.6893x; 1.9325x over previous
import functools

import jax
import jax.numpy as jnp
from jax import lax
from jax.experimental import pallas as pl
from jax.experimental.pallas import tpu as pltpu
from jax.experimental.pallas import tpu_sc as plsc

N = 4_000_000
NN = 2_000_000
NM = 1_800_000
NROWS = 2048

LANES = 128
R = 744
BLK = R * LANES
POS_ROWS = N // LANES
FRAME_LO = 1_999_872
FRAME_LEN = 1_809_408
FRAME_HI = FRAME_LO + FRAME_LEN
FRAME_ROWS = FRAME_LEN // LANES
FRAME_BLKS = FRAME_LEN // BLK
FRAME_ROW0 = FRAME_LO // LANES
FRAME_BLK0 = FRAME_LO // BLK
MASK_LEFT_PAD = NN - FRAME_LO

NC, NS, L = 2, 16, 16
NW = NC * NS

CC = 15_872
NA = FRAME_LO // CC
NCOPY = NA + (N - FRAME_HI) // CC
TAIL = (N - FRAME_HI) - (NCOPY - NA) * CC
TAIL_OFF = N - TAIL
NV = -(-NCOPY // NW)
NBUF = 4


def _sc_copy_body(pos_hbm, out_hbm, shared, isems, osems):
    cid = lax.axis_index("c")
    sid = lax.axis_index("s")
    wid = sid * NC + cid

    def descs(u):
        j = wid + u * NW
        b = u % NBUF
        base = jnp.where(j < NA, j * CC, FRAME_HI + (j - NA) * CC)
        row = shared.at[pl.ds((sid * NBUF + b) * CC, CC)]
        in_d = pltpu.make_async_copy(
            pos_hbm.at[pl.ds(base, CC)], row, isems[b])
        out_d = pltpu.make_async_copy(
            row, out_hbm.at[pl.ds(base, CC)], osems[b])
        return j, in_d, out_d

    def issue_in(u):
        j, in_d, _ = descs(u)

        @pl.when(j < NCOPY)
        def _():
            in_d.start()

    issue_in(0)
    for u in range(NV):
        j, in_d, out_d = descs(u)
        if u >= NBUF - 1:
            jp, _, out_dp = descs(u - (NBUF - 1))

            @pl.when(jp < NCOPY)
            def _():
                out_dp.wait()
        if u + 1 < NV:
            issue_in(u + 1)

        @pl.when(j < NCOPY)
        def _():
            in_d.wait()
            out_d.start()
    for u in range(max(NV - (NBUF - 1), 0), NV):
        j, _, out_d = descs(u)

        @pl.when(j < NCOPY)
        def _():
            out_d.wait()

    @pl.when(wid == 0)
    def _tail():
        tb = shared.at[pl.ds(0, TAIL)]
        pltpu.sync_copy(pos_hbm.at[pl.ds(TAIL_OFF, TAIL)], tb)
        pltpu.sync_copy(tb, out_hbm.at[pl.ds(TAIL_OFF, TAIL)])


@functools.lru_cache(maxsize=1)
def _build_sc():
    mesh = plsc.VectorSubcoreMesh(core_axis_name="c", subcore_axis_name="s",
                                  num_cores=NC, num_subcores=NS)
    return pl.kernel(
        _sc_copy_body,
        out_type=jax.ShapeDtypeStruct((N,), jnp.float32),
        mesh=mesh,
        scratch_types=[
            pltpu.VMEM_SHARED((NS * NBUF * CC,), jnp.float32),
            [pltpu.SemaphoreType.DMA for _ in range(NBUF)],
            [pltpu.SemaphoreType.DMA for _ in range(NBUF)],
        ],
        compiler_params=pltpu.CompilerParams(needs_layout_passes=False),
    )


def _tc_quant_body(acc_ref, pos_ref, mask_ref, out_ref):
    del acc_ref
    x = pos_ref[...]
    q = jnp.minimum(jnp.round(x), float(NROWS - 1))
    out_ref[...] = jnp.where(mask_ref[...], q, x)


def _tc_quantize(out1, pos2, maskp):
    return pl.pallas_call(
        _tc_quant_body,
        grid=(FRAME_BLKS,),
        in_specs=[
            pl.BlockSpec((8, LANES), lambda i: (0, 0)),
            pl.BlockSpec((R, LANES), lambda i: (i + FRAME_BLK0, 0)),
            pl.BlockSpec((R, LANES), lambda i: (i, 0)),
        ],
        out_specs=pl.BlockSpec((R, LANES), lambda i: (i + FRAME_BLK0, 0)),
        out_shape=jax.ShapeDtypeStruct((POS_ROWS, LANES), jnp.float32),
        input_output_aliases={0: 0},
    )(out1, pos2, maskp)


def kernel(pos, mask):
    out1 = _build_sc()(pos)
    maskp = jnp.concatenate([
        jnp.zeros((MASK_LEFT_PAD,), jnp.bool_),
        mask,
        jnp.zeros((FRAME_LEN - MASK_LEFT_PAD - NM,), jnp.bool_),
    ]).reshape(FRAME_ROWS, LANES)
    out = _tc_quantize(out1.reshape(POS_ROWS, LANES),
                       pos.reshape(POS_ROWS, LANES), maskp)
    return out.reshape(N)
